# baseline (speedup 1.0000x reference)
import jax
import jax.numpy as jnp
from jax import lax
from jax.experimental import pallas as pl

_NMS_T = 0.6
_NPAD = 5120
_K = 2048
_KV = 2000
_B = 256
_OC = 256
_RC = 512
_OUTP = 1024
_F32 = jnp.float32


def _csum_incl(x):
    sh = 1
    while sh < _K:
        x = x + jnp.concatenate(
            [jnp.zeros((1, sh), _F32), x[:, : _K - sh]], axis=1)
        sh *= 2
    return x


def _body(b5_ref, b5t_ref, out_ref):
    s_row = b5t_ref[4:5, :]
    i_col = lax.broadcasted_iota(jnp.int32, (1, _NPAD), 1)
    rank = jnp.zeros((1, _NPAD), _F32)
    for jc in range(_NPAD // _RC):
        sj = b5_ref[jc * _RC:(jc + 1) * _RC, 4:5]
        jidx = lax.broadcasted_iota(jnp.int32, (_RC, 1), 0) + jc * _RC
        beats = (sj > s_row) | ((sj == s_row) & (jidx < i_col))
        rank = rank + jnp.sum(beats.astype(_F32), axis=0, keepdims=True)

    b5 = b5_ref[:, :]
    b5t = b5t_ref[:, :]
    t5_chunks, t5c_chunks = [], []
    for oc in range(_K // _OC):
        jout = (lax.broadcasted_iota(jnp.int32, (_OC, 1), 0)
                + oc * _OC).astype(_F32)
        oh = (rank == jout).astype(_F32)
        t5_chunks.append(lax.dot_general(
            oh, b5, (((1,), (0,)), ((), ())),
            precision=lax.Precision.HIGHEST, preferred_element_type=_F32))
        t5c_chunks.append(lax.dot_general(
            b5t, oh, (((1,), (1,)), ((), ())),
            precision=lax.Precision.HIGHEST, preferred_element_type=_F32))
    t5 = jnp.concatenate(t5_chunks, axis=0)
    t5c = jnp.concatenate(t5c_chunks, axis=1)

    boxcol = lax.broadcasted_iota(jnp.int32, (_K, 5), 1) < 4
    t5s = jnp.where(boxcol, t5 * 1024.0, t5)
    boxrow = lax.broadcasted_iota(jnp.int32, (5, _K), 0) < 4
    t5cs = jnp.where(boxrow, t5c * 1024.0, t5c)

    x1c = jnp.minimum(t5cs[0:1, :], t5cs[2:3, :])
    x2c = jnp.maximum(t5cs[0:1, :], t5cs[2:3, :])
    y1c = jnp.minimum(t5cs[1:2, :], t5cs[3:4, :])
    y2c = jnp.maximum(t5cs[1:2, :], t5cs[3:4, :])
    areac = (x2c - x1c) * (y2c - y1c)

    col_k = lax.broadcasted_iota(jnp.int32, (1, _K), 1)
    eye_b = (lax.broadcasted_iota(jnp.int32, (_B, _B), 0)
             == lax.broadcasted_iota(jnp.int32, (_B, _B), 1)).astype(_F32)

    keep = jnp.ones((1, _K), _F32)
    for blk in range(_K // _B):
        r0 = blk * _B
        bb = t5s[r0:r0 + _B, :]
        x1b = jnp.minimum(bb[:, 0:1], bb[:, 2:3])
        x2b = jnp.maximum(bb[:, 0:1], bb[:, 2:3])
        y1b = jnp.minimum(bb[:, 1:2], bb[:, 3:4])
        y2b = jnp.maximum(bb[:, 1:2], bb[:, 3:4])
        areab = (x2b - x1b) * (y2b - y1b)

        kw = _K - r0
        ix1 = jnp.maximum(x1b, x1c[:, r0:])
        iy1 = jnp.maximum(y1b, y1c[:, r0:])
        ix2 = jnp.minimum(x2b, x2c[:, r0:])
        iy2 = jnp.minimum(y2b, y2c[:, r0:])
        iw = jnp.maximum(ix2 - ix1, 0.0)
        ih = jnp.maximum(iy2 - iy1, 0.0)
        inter = iw * ih
        union = areab + areac[:, r0:] - inter
        iou = inter / jnp.maximum(union, 1e-9)
        hot = iou > _NMS_T

        row_g = lax.broadcasted_iota(jnp.int32, (_B, 1), 0) + r0
        sfull = (hot & (col_k[:, r0:] > row_g)).astype(_F32)

        hot_d = hot[:, :_B]
        row_b = lax.broadcasted_iota(jnp.int32, (_B, 1), 0)
        col_b = lax.broadcasted_iota(jnp.int32, (1, _B), 1)
        d_up = (hot_d & (col_b > row_b)).astype(_F32)
        d_lo = (hot_d & (row_b > col_b)).astype(_F32)

        kb0_r = keep[:, r0:r0 + _B]
        kb0_c = lax.dot_general(
            eye_b, kb0_r, (((1,), (1,)), ((), ())),
            preferred_element_type=_F32)

        def cond(c):
            return c[2] & (c[3] < _B + 2)

        def step(c, kb0_r=kb0_r, kb0_c=kb0_c, d_up=d_up, d_lo=d_lo):
            kr, kc, _, it = c
            supp_r = jnp.max(d_up * kc, axis=0, keepdims=True)
            supp_c = jnp.max(d_lo * kr, axis=1, keepdims=True)
            kr2 = kb0_r * (1.0 - supp_r)
            kc2 = kb0_c * (1.0 - supp_c)
            return kr2, kc2, jnp.any(kr2 != kr), it + 1

        kr_f, kc_f, _, _ = lax.while_loop(
            cond, step, (kb0_r, kb0_c, jnp.bool_(True), jnp.int32(0)))

        supp = jnp.max(sfull * kc_f, axis=0, keepdims=True)
        if r0 + _B < _K:
            tail = jnp.concatenate([kr_f, keep[:, r0 + _B:]], axis=1)
        else:
            tail = kr_f
        tail = tail * (1.0 - supp)
        keep = tail if r0 == 0 else jnp.concatenate(
            [keep[:, :r0], tail], axis=1)

    valid = (col_k < _KV).astype(_F32)
    kv = keep * valid
    sv = valid * (1.0 - keep)
    ck = _csum_incl(kv) - kv
    cs = _csum_incl(sv) - sv
    nk = jnp.sum(kv)
    pos = jnp.where(kv > 0, ck, nk + cs)
    pos = jnp.where(valid > 0, pos, 4096.0)

    oidx = lax.broadcasted_iota(jnp.int32, (_OUTP, 1), 0).astype(_F32)
    ohf = (pos == oidx).astype(_F32)
    out5 = lax.dot_general(
        ohf, t5s, (((1,), (0,)), ((), ())),
        precision=lax.Precision.HIGHEST, preferred_element_type=_F32)
    kept_out = (oidx < nk).astype(_F32)
    score_col = lax.broadcasted_iota(jnp.int32, (_OUTP, 5), 1) == 4
    out_ref[:, :] = jnp.where(score_col, out5 * kept_out, out5)


def kernel(boxes, scores):
    boxes = boxes.astype(_F32)
    scores = scores.astype(_F32)
    n = boxes.shape[0]
    b5 = jnp.concatenate([boxes, scores[:, None]], axis=1)
    pad = _NPAD - n
    padrow = jnp.concatenate(
        [jnp.zeros((pad, 4), _F32), jnp.full((pad, 1), -1.0, _F32)], axis=1)
    b5p = jnp.concatenate([b5, padrow], axis=0)
    b5t = b5p.T
    out = pl.pallas_call(
        _body,
        out_shape=jax.ShapeDtypeStruct((_OUTP, 5), _F32),
    )(b5p, b5t)
    return out[:1000]


# speedup vs baseline: 1.3700x; 1.3700x over previous
import jax
import jax.numpy as jnp
from jax import lax
from jax.experimental import pallas as pl

_NMS_T = 0.6
_NPAD = 5120
_K = 2048
_KV = 2000
_B = 256
_OC = 256
_RC = 512
_OUTP = 1024
_F32 = jnp.float32


def _split3(x):
    p0 = x.astype(jnp.bfloat16)
    r1 = x - p0.astype(_F32)
    p1 = r1.astype(jnp.bfloat16)
    p2 = (r1 - p1.astype(_F32)).astype(jnp.bfloat16)
    return p0, p1, p2


def _exact_dot(oh, parts, dnums):
    ohb = oh.astype(jnp.bfloat16)
    acc = None
    for p in parts:
        d = lax.dot_general(ohb, p, dnums, preferred_element_type=_F32)
        acc = d if acc is None else acc + d
    return acc


def _exact_dot_r(parts, oh, dnums):
    ohb = oh.astype(jnp.bfloat16)
    acc = None
    for p in parts:
        d = lax.dot_general(p, ohb, dnums, preferred_element_type=_F32)
        acc = d if acc is None else acc + d
    return acc


def _csum_incl(x):
    sh = 1
    while sh < _K:
        x = x + jnp.concatenate(
            [jnp.zeros((1, sh), _F32), x[:, : _K - sh]], axis=1)
        sh *= 2
    return x


def _body(b5_ref, b5t_ref, out_ref):
    s_row = b5t_ref[4:5, :]
    i_col = lax.broadcasted_iota(jnp.int32, (1, _NPAD), 1)
    rank = jnp.zeros((1, _NPAD), _F32)
    for jc in range(_NPAD // _RC):
        sj = b5_ref[jc * _RC:(jc + 1) * _RC, 4:5]
        jidx = lax.broadcasted_iota(jnp.int32, (_RC, 1), 0) + jc * _RC
        beats = (sj > s_row) | ((sj == s_row) & (jidx < i_col))
        rank = rank + jnp.sum(beats.astype(_F32), axis=0, keepdims=True)

    b5_parts = _split3(b5_ref[:, :])
    b5t_parts = _split3(b5t_ref[:, :])
    t5_chunks, t5c_chunks = [], []
    for oc in range(_K // _OC):
        jout = (lax.broadcasted_iota(jnp.int32, (_OC, 1), 0)
                + oc * _OC).astype(_F32)
        oh = (rank == jout).astype(_F32)
        t5_chunks.append(_exact_dot(
            oh, b5_parts, (((1,), (0,)), ((), ()))))
        t5c_chunks.append(_exact_dot_r(
            b5t_parts, oh, (((1,), (1,)), ((), ()))))
    t5 = jnp.concatenate(t5_chunks, axis=0)
    t5c = jnp.concatenate(t5c_chunks, axis=1)

    boxcol = lax.broadcasted_iota(jnp.int32, (_K, 5), 1) < 4
    t5s = jnp.where(boxcol, t5 * 1024.0, t5)
    boxrow = lax.broadcasted_iota(jnp.int32, (5, _K), 0) < 4
    t5cs = jnp.where(boxrow, t5c * 1024.0, t5c)

    x1c = jnp.minimum(t5cs[0:1, :], t5cs[2:3, :])
    x2c = jnp.maximum(t5cs[0:1, :], t5cs[2:3, :])
    y1c = jnp.minimum(t5cs[1:2, :], t5cs[3:4, :])
    y2c = jnp.maximum(t5cs[1:2, :], t5cs[3:4, :])
    areac = (x2c - x1c) * (y2c - y1c)

    col_k = lax.broadcasted_iota(jnp.int32, (1, _K), 1)
    eye_b = (lax.broadcasted_iota(jnp.int32, (_B, _B), 0)
             == lax.broadcasted_iota(jnp.int32, (_B, _B), 1)).astype(_F32)

    keep = jnp.ones((1, _K), _F32)
    for blk in range(_K // _B):
        r0 = blk * _B
        bb = t5s[r0:r0 + _B, :]
        x1b = jnp.minimum(bb[:, 0:1], bb[:, 2:3])
        x2b = jnp.maximum(bb[:, 0:1], bb[:, 2:3])
        y1b = jnp.minimum(bb[:, 1:2], bb[:, 3:4])
        y2b = jnp.maximum(bb[:, 1:2], bb[:, 3:4])
        areab = (x2b - x1b) * (y2b - y1b)

        kw = _K - r0
        ix1 = jnp.maximum(x1b, x1c[:, r0:])
        iy1 = jnp.maximum(y1b, y1c[:, r0:])
        ix2 = jnp.minimum(x2b, x2c[:, r0:])
        iy2 = jnp.minimum(y2b, y2c[:, r0:])
        iw = jnp.maximum(ix2 - ix1, 0.0)
        ih = jnp.maximum(iy2 - iy1, 0.0)
        inter = iw * ih
        union = areab + areac[:, r0:] - inter
        iou = inter / jnp.maximum(union, 1e-9)
        hot = iou > _NMS_T

        row_g = lax.broadcasted_iota(jnp.int32, (_B, 1), 0) + r0
        sfull = (hot & (col_k[:, r0:] > row_g)).astype(_F32)

        hot_d = hot[:, :_B]
        row_b = lax.broadcasted_iota(jnp.int32, (_B, 1), 0)
        col_b = lax.broadcasted_iota(jnp.int32, (1, _B), 1)
        d_up = (hot_d & (col_b > row_b)).astype(_F32)
        d_lo = (hot_d & (row_b > col_b)).astype(_F32)

        kb0_r = keep[:, r0:r0 + _B]
        kb0_c = lax.dot_general(
            eye_b, kb0_r, (((1,), (1,)), ((), ())),
            preferred_element_type=_F32)

        def cond(c):
            return c[2] & (c[3] < _B + 2)

        def step(c, kb0_r=kb0_r, kb0_c=kb0_c, d_up=d_up, d_lo=d_lo):
            kr, kc, _, it = c
            supp_r = jnp.max(d_up * kc, axis=0, keepdims=True)
            supp_c = jnp.max(d_lo * kr, axis=1, keepdims=True)
            kr2 = kb0_r * (1.0 - supp_r)
            kc2 = kb0_c * (1.0 - supp_c)
            return kr2, kc2, jnp.any(kr2 != kr), it + 1

        kr_f, kc_f, _, _ = lax.while_loop(
            cond, step, (kb0_r, kb0_c, jnp.bool_(True), jnp.int32(0)))

        supp = jnp.max(sfull * kc_f, axis=0, keepdims=True)
        if r0 + _B < _K:
            tail = jnp.concatenate([kr_f, keep[:, r0 + _B:]], axis=1)
        else:
            tail = kr_f
        tail = tail * (1.0 - supp)
        keep = tail if r0 == 0 else jnp.concatenate(
            [keep[:, :r0], tail], axis=1)

    valid = (col_k < _KV).astype(_F32)
    kv = keep * valid
    sv = valid * (1.0 - keep)
    ck = _csum_incl(kv) - kv
    cs = _csum_incl(sv) - sv
    nk = jnp.sum(kv)
    pos = jnp.where(kv > 0, ck, nk + cs)
    pos = jnp.where(valid > 0, pos, 4096.0)

    oidx = lax.broadcasted_iota(jnp.int32, (_OUTP, 1), 0).astype(_F32)
    ohf = (pos == oidx).astype(_F32)
    out5 = _exact_dot(ohf, _split3(t5s), (((1,), (0,)), ((), ())))
    kept_out = (oidx < nk).astype(_F32)
    score_col = lax.broadcasted_iota(jnp.int32, (_OUTP, 5), 1) == 4
    out_ref[:, :] = jnp.where(score_col, out5 * kept_out, out5)


def kernel(boxes, scores):
    boxes = boxes.astype(_F32)
    scores = scores.astype(_F32)
    n = boxes.shape[0]
    b5 = jnp.concatenate([boxes, scores[:, None]], axis=1)
    pad = _NPAD - n
    padrow = jnp.concatenate(
        [jnp.zeros((pad, 4), _F32), jnp.full((pad, 1), -1.0, _F32)], axis=1)
    b5p = jnp.concatenate([b5, padrow], axis=0)
    b5t = b5p.T
    out = pl.pallas_call(
        _body,
        out_shape=jax.ShapeDtypeStruct((_OUTP, 5), _F32),
    )(b5p, b5t)
    return out[:1000]


# speedup vs baseline: 1.4501x; 1.0585x over previous
import jax
import jax.numpy as jnp
from jax import lax
from jax.experimental import pallas as pl

_NMS_T = 0.6
_NPAD = 5120
_K = 2048
_KV = 2000
_B = 256
_OC = 256
_RC = 512
_OUTP = 1024
_F32 = jnp.float32


def _split3(x):
    p0 = x.astype(jnp.bfloat16)
    r1 = x - p0.astype(_F32)
    p1 = r1.astype(jnp.bfloat16)
    p2 = (r1 - p1.astype(_F32)).astype(jnp.bfloat16)
    return p0, p1, p2


def _exact_dot(oh, parts, dnums):
    ohb = oh.astype(jnp.bfloat16)
    acc = None
    for p in parts:
        d = lax.dot_general(ohb, p, dnums, preferred_element_type=_F32)
        acc = d if acc is None else acc + d
    return acc


def _exact_dot_r(parts, oh, dnums):
    ohb = oh.astype(jnp.bfloat16)
    acc = None
    for p in parts:
        d = lax.dot_general(p, ohb, dnums, preferred_element_type=_F32)
        acc = d if acc is None else acc + d
    return acc


def _csum_incl(x):
    sh = 1
    while sh < _K:
        x = x + jnp.concatenate(
            [jnp.zeros((1, sh), _F32), x[:, : _K - sh]], axis=1)
        sh *= 2
    return x


def _body(b5_ref, b5t_ref, out_ref):
    s_row = b5t_ref[4:5, :]
    i_col = lax.broadcasted_iota(jnp.int32, (1, _NPAD), 1)
    ones_rc = jnp.ones((1, _RC), jnp.bfloat16)
    rank = jnp.zeros((1, _NPAD), _F32)
    for jc in range(_NPAD // _RC):
        sj = b5_ref[jc * _RC:(jc + 1) * _RC, 4:5]
        jidx = lax.broadcasted_iota(jnp.int32, (_RC, 1), 0) + jc * _RC
        beats = (sj > s_row) | ((sj == s_row) & (jidx < i_col))
        rank = rank + lax.dot_general(
            ones_rc, beats.astype(jnp.bfloat16), (((1,), (0,)), ((), ())),
            preferred_element_type=_F32)

    b5_parts = _split3(b5_ref[:, :])
    b5t_parts = _split3(b5t_ref[:, :])
    t5_chunks, t5c_chunks = [], []
    for oc in range(_K // _OC):
        jout = (lax.broadcasted_iota(jnp.int32, (_OC, 1), 0)
                + oc * _OC).astype(_F32)
        oh = (rank == jout).astype(_F32)
        t5_chunks.append(_exact_dot(
            oh, b5_parts, (((1,), (0,)), ((), ()))))
        t5c_chunks.append(_exact_dot_r(
            b5t_parts, oh, (((1,), (1,)), ((), ()))))
    t5 = jnp.concatenate(t5_chunks, axis=0)
    t5c = jnp.concatenate(t5c_chunks, axis=1)

    boxcol = lax.broadcasted_iota(jnp.int32, (_K, 5), 1) < 4
    t5s = jnp.where(boxcol, t5 * 1024.0, t5)
    boxrow = lax.broadcasted_iota(jnp.int32, (5, _K), 0) < 4
    t5cs = jnp.where(boxrow, t5c * 1024.0, t5c)

    x1c = jnp.minimum(t5cs[0:1, :], t5cs[2:3, :])
    x2c = jnp.maximum(t5cs[0:1, :], t5cs[2:3, :])
    y1c = jnp.minimum(t5cs[1:2, :], t5cs[3:4, :])
    y2c = jnp.maximum(t5cs[1:2, :], t5cs[3:4, :])
    areac = (x2c - x1c) * (y2c - y1c)

    col_k = lax.broadcasted_iota(jnp.int32, (1, _K), 1)
    eye_b = (lax.broadcasted_iota(jnp.int32, (_B, _B), 0)
             == lax.broadcasted_iota(jnp.int32, (_B, _B), 1)).astype(_F32)

    keep = jnp.ones((1, _K), _F32)
    for blk in range(_K // _B):
        r0 = blk * _B
        bb = t5s[r0:r0 + _B, :]
        x1b = jnp.minimum(bb[:, 0:1], bb[:, 2:3])
        x2b = jnp.maximum(bb[:, 0:1], bb[:, 2:3])
        y1b = jnp.minimum(bb[:, 1:2], bb[:, 3:4])
        y2b = jnp.maximum(bb[:, 1:2], bb[:, 3:4])
        areab = (x2b - x1b) * (y2b - y1b)

        kw = _K - r0
        ix1 = jnp.maximum(x1b, x1c[:, r0:])
        iy1 = jnp.maximum(y1b, y1c[:, r0:])
        ix2 = jnp.minimum(x2b, x2c[:, r0:])
        iy2 = jnp.minimum(y2b, y2c[:, r0:])
        iw = jnp.maximum(ix2 - ix1, 0.0)
        ih = jnp.maximum(iy2 - iy1, 0.0)
        inter = iw * ih
        union = areab + areac[:, r0:] - inter
        iou = inter / jnp.maximum(union, 1e-9)
        hot = iou > _NMS_T

        row_g = lax.broadcasted_iota(jnp.int32, (_B, 1), 0) + r0
        sfull = (hot & (col_k[:, r0:] > row_g)).astype(_F32)

        hot_d = hot[:, :_B]
        row_b = lax.broadcasted_iota(jnp.int32, (_B, 1), 0)
        col_b = lax.broadcasted_iota(jnp.int32, (1, _B), 1)
        d_up = (hot_d & (col_b > row_b)).astype(_F32)
        d_lo = (hot_d & (row_b > col_b)).astype(_F32)

        kb0_r = keep[:, r0:r0 + _B]
        kb0_c = lax.dot_general(
            eye_b, kb0_r, (((1,), (1,)), ((), ())),
            preferred_element_type=_F32)

        def cond(c):
            return c[2] & (c[3] < _B + 2)

        def step(c, kb0_r=kb0_r, kb0_c=kb0_c, d_up=d_up, d_lo=d_lo):
            kr, kc, _, it = c
            supp_r = jnp.max(d_up * kc, axis=0, keepdims=True)
            supp_c = jnp.max(d_lo * kr, axis=1, keepdims=True)
            kr2 = kb0_r * (1.0 - supp_r)
            kc2 = kb0_c * (1.0 - supp_c)
            return kr2, kc2, jnp.any(kr2 != kr), it + 1

        kr_f, kc_f, _, _ = lax.while_loop(
            cond, step, (kb0_r, kb0_c, jnp.bool_(True), jnp.int32(0)))

        cnt = lax.dot_general(
            kr_f.astype(jnp.bfloat16), sfull.astype(jnp.bfloat16),
            (((1,), (0,)), ((), ())), preferred_element_type=_F32)
        supp = (cnt > 0.0).astype(_F32)
        if r0 + _B < _K:
            tail = jnp.concatenate([kr_f, keep[:, r0 + _B:]], axis=1)
        else:
            tail = kr_f
        tail = tail * (1.0 - supp)
        keep = tail if r0 == 0 else jnp.concatenate(
            [keep[:, :r0], tail], axis=1)

    valid = (col_k < _KV).astype(_F32)
    kv = keep * valid
    sv = valid * (1.0 - keep)
    ck = _csum_incl(kv) - kv
    cs = _csum_incl(sv) - sv
    nk = jnp.sum(kv)
    pos = jnp.where(kv > 0, ck, nk + cs)
    pos = jnp.where(valid > 0, pos, 4096.0)

    oidx = lax.broadcasted_iota(jnp.int32, (_OUTP, 1), 0).astype(_F32)
    ohf = (pos == oidx).astype(_F32)
    out5 = _exact_dot(ohf, _split3(t5s), (((1,), (0,)), ((), ())))
    kept_out = (oidx < nk).astype(_F32)
    score_col = lax.broadcasted_iota(jnp.int32, (_OUTP, 5), 1) == 4
    out_ref[:, :] = jnp.where(score_col, out5 * kept_out, out5)


def kernel(boxes, scores):
    boxes = boxes.astype(_F32)
    scores = scores.astype(_F32)
    n = boxes.shape[0]
    b5 = jnp.concatenate([boxes, scores[:, None]], axis=1)
    pad = _NPAD - n
    padrow = jnp.concatenate(
        [jnp.zeros((pad, 4), _F32), jnp.full((pad, 1), -1.0, _F32)], axis=1)
    b5p = jnp.concatenate([b5, padrow], axis=0)
    b5t = b5p.T
    out = pl.pallas_call(
        _body,
        out_shape=jax.ShapeDtypeStruct((_OUTP, 5), _F32),
    )(b5p, b5t)
    return out[:1000]
